# baseline (speedup 1.0000x reference)
import functools

import jax
import jax.numpy as jnp
from jax import lax
from jax.experimental import pallas as pl
from jax.experimental.pallas import tpu as pltpu
from jax.experimental.pallas import tpu_sc as plsc

N = 10000
E = 320000
D = 128
H = 16
C = 40

NC = 2
NS = 16
KB = 125
BB = 80
KT = NC * KB
RPT = N // NS
P = 5

_mesh = plsc.VectorSubcoreMesh(core_axis_name="c", subcore_axis_name="s")
_sc_params = pltpu.CompilerParams(use_tc_tiling_on_sc=False,
                                  needs_layout_passes=False)


def _rsqrt16(x):
    i = plsc.bitcast(x, jnp.int32)
    i = 0x5F3759DF - lax.shift_right_logical(i, 1)
    y = plsc.bitcast(i, jnp.float32)
    xh = 0.5 * x
    for _ in range(3):
        y = y * (1.5 - xh * y * y)
    return y


def _sc_a(h1, src_blk, dst_blk, ones_rows, zero_rows):
    @functools.partial(
        pl.kernel,
        out_type=[
            jax.ShapeDtypeStruct((NC, N, H), jnp.float32),
            jax.ShapeDtypeStruct((NC, N, H), jnp.float32),
            jax.ShapeDtypeStruct((NC, N, H), jnp.float32),
        ],
        mesh=_mesh,
        compiler_params=_sc_params,
        scratch_types=[
            pltpu.VMEM((KT, BB), jnp.int32),
            pltpu.VMEM((KT, BB), jnp.int32),
            pltpu.VMEM((BB, H), jnp.float32),
            pltpu.VMEM((RPT, H), jnp.float32),
            pltpu.VMEM((RPT, H), jnp.float32),
            pltpu.VMEM((RPT, H), jnp.float32),
            pltpu.VMEM((P, BB, H), jnp.float32),
            pltpu.VMEM_SHARED((N, H), jnp.float32),
            pltpu.VMEM_SHARED((N, H), jnp.float32),
            pltpu.SemaphoreType.DMA((P,)),
        ],
    )
    def sc_a(h1_hbm, src_hbm, dst_hbm, ones_hbm, zeros_hbm,
             s1_hbm, g1_hbm, dis_hbm,
             sidx, didx, ones_v, zv, va, vb, rows, acc, gtab, sem):
        c = lax.axis_index("c")
        s = lax.axis_index("s")
        sl = pl.ds(s * RPT, RPT)
        pltpu.sync_copy(zeros_hbm, zv)
        pltpu.sync_copy(zv, acc.at[sl])
        pltpu.sync_copy(ones_hbm, ones_v)
        pltpu.sync_copy(src_hbm.at[s], sidx)
        pltpu.sync_copy(dst_hbm.at[s], didx)
        plsc.subcore_barrier()

        def deg_body(jj, _):
            for b in range(P):
                j = jj * P + b
                pltpu.async_copy(ones_v, acc.at[didx.at[j]], sem.at[b],
                                 add=True)

                @pl.when(jj >= 1)
                def _wait():
                    pltpu.make_async_copy(
                        ones_v, acc.at[didx.at[j - P]], sem.at[b]).wait()

            return _

        lax.fori_loop(0, KT // P, deg_body, None)
        for b in range(P):
            pltpu.make_async_copy(
                ones_v, acc.at[didx.at[KT - P + b]], sem.at[b]).wait()
        plsc.subcore_barrier()

        pltpu.sync_copy(acc.at[sl], va)
        pltpu.sync_copy(h1_hbm.at[sl], vb)

        def dis_row(i, _):
            dis = _rsqrt16(va[i, :] + 1.0)
            va[i, :] = dis
            vb[i, :] = dis * vb[i, :]
            return _

        lax.fori_loop(0, RPT, dis_row, None)
        pltpu.sync_copy(va, dis_hbm.at[c, sl])
        pltpu.sync_copy(vb, g1_hbm.at[c, sl])
        pltpu.sync_copy(vb, gtab.at[sl])
        pltpu.sync_copy(zv, acc.at[sl])
        plsc.subcore_barrier()

        q0 = c * KB
        for b in range(P):
            pltpu.async_copy(gtab.at[sidx.at[q0 + b]], rows.at[b], sem.at[b])

        def mp_body(jj, _):
            for b in range(P):
                j = q0 + jj * P + b
                pltpu.make_async_copy(
                    gtab.at[sidx.at[j]], rows.at[b], sem.at[b]).wait()
                pltpu.sync_copy(rows.at[b], acc.at[didx.at[j]], add=True)

                @pl.when(jj * P + b + P < KB)
                def _next():
                    pltpu.async_copy(
                        gtab.at[sidx.at[j + P]], rows.at[b], sem.at[b])

            return _

        lax.fori_loop(0, KB // P, mp_body, None)
        plsc.subcore_barrier()
        pltpu.sync_copy(acc.at[sl], va)
        pltpu.sync_copy(va, s1_hbm.at[c, sl])

    return sc_a(h1, src_blk, dst_blk, ones_rows, zero_rows)


def _sc_b(s1, g1, dis, b1row, src_blk, dst_blk, zero_rows):
    @functools.partial(
        pl.kernel,
        out_type=[
            jax.ShapeDtypeStruct((NC, N, H), jnp.float32),
            jax.ShapeDtypeStruct((NC, N, H), jnp.float32),
        ],
        mesh=_mesh,
        compiler_params=_sc_params,
        scratch_types=[
            pltpu.VMEM((KT, BB), jnp.int32),
            pltpu.VMEM((KT, BB), jnp.int32),
            pltpu.VMEM((16, H), jnp.float32),
            pltpu.VMEM((RPT, H), jnp.float32),
            pltpu.VMEM((RPT, H), jnp.float32),
            pltpu.VMEM((RPT, H), jnp.float32),
            pltpu.VMEM((RPT, H), jnp.float32),
            pltpu.VMEM((P, BB, H), jnp.float32),
            pltpu.VMEM_SHARED((N, H), jnp.float32),
            pltpu.VMEM_SHARED((N, H), jnp.float32),
            pltpu.SemaphoreType.DMA((P,)),
        ],
    )
    def sc_b(s1_hbm, g1_hbm, dis_hbm, b1_hbm, src_hbm, dst_hbm, zeros_hbm,
             s2_hbm, g2_hbm,
             sidx, didx, b1v, zv, va, vb, vc, rows, acc, gtab, sem):
        c = lax.axis_index("c")
        s = lax.axis_index("s")
        sl = pl.ds(s * RPT, RPT)
        pltpu.sync_copy(zeros_hbm, zv)
        pltpu.sync_copy(src_hbm.at[s], sidx)
        pltpu.sync_copy(dst_hbm.at[s], didx)
        pltpu.sync_copy(b1_hbm, b1v)

        pltpu.sync_copy(s1_hbm.at[0, sl], va)
        pltpu.sync_copy(s1_hbm.at[1, sl], vb)
        pltpu.sync_copy(g1_hbm.at[c, sl], vc)

        def s1_row(i, _):
            va[i, :] = va[i, :] + vb[i, :] + vc[i, :]
            return _

        lax.fori_loop(0, RPT, s1_row, None)
        pltpu.sync_copy(dis_hbm.at[c, sl], vb)

        def g2_row(i, _):
            agg = vb[i, :] * va[i, :] + b1v[0, :]
            va[i, :] = vb[i, :] * jnp.maximum(agg, 0.0)
            return _

        lax.fori_loop(0, RPT, g2_row, None)
        pltpu.sync_copy(va, g2_hbm.at[c, sl])
        pltpu.sync_copy(va, gtab.at[sl])
        pltpu.sync_copy(zv, acc.at[sl])
        plsc.subcore_barrier()

        q0 = c * KB
        for b in range(P):
            pltpu.async_copy(gtab.at[sidx.at[q0 + b]], rows.at[b], sem.at[b])

        def mp_body(jj, _):
            for b in range(P):
                j = q0 + jj * P + b
                pltpu.make_async_copy(
                    gtab.at[sidx.at[j]], rows.at[b], sem.at[b]).wait()
                pltpu.sync_copy(rows.at[b], acc.at[didx.at[j]], add=True)

                @pl.when(jj * P + b + P < KB)
                def _next():
                    pltpu.async_copy(
                        gtab.at[sidx.at[j + P]], rows.at[b], sem.at[b])

            return _

        lax.fori_loop(0, KB // P, mp_body, None)
        plsc.subcore_barrier()
        pltpu.sync_copy(acc.at[sl], va)
        pltpu.sync_copy(va, s2_hbm.at[c, sl])

    return sc_b(s1, g1, dis, b1row, src_blk, dst_blk, zero_rows)


_R = 1000


def _tc_matmul(x, w1):
    def body(x_ref, w_ref, o_ref):
        o_ref[...] = jnp.dot(x_ref[...], w_ref[...],
                             preferred_element_type=jnp.float32)

    return pl.pallas_call(
        body,
        grid=(N // _R,),
        in_specs=[
            pl.BlockSpec((_R, D), lambda i: (i, 0)),
            pl.BlockSpec((D, H), lambda i: (0, 0)),
        ],
        out_specs=pl.BlockSpec((_R, H), lambda i: (i, 0)),
        out_shape=jax.ShapeDtypeStruct((N, H), jnp.float32),
    )(x, w1)


def _tc_final(s2, g2, dis, w2, b2):
    def body(s_ref, g_ref, dis_ref, w_ref, b_ref, o_ref):
        pre = jnp.dot(s_ref[0] + s_ref[1] + g_ref[...], w_ref[...],
                      preferred_element_type=jnp.float32)
        logits = dis_ref[:, :1] * pre + b_ref[...]
        m = jnp.max(logits, axis=1, keepdims=True)
        t = logits - m
        lse = jnp.log(jnp.sum(jnp.exp(t), axis=1, keepdims=True))
        o_ref[...] = t - lse

    return pl.pallas_call(
        body,
        grid=(N // _R,),
        in_specs=[
            pl.BlockSpec((NC, _R, H), lambda i: (0, i, 0)),
            pl.BlockSpec((_R, H), lambda i: (i, 0)),
            pl.BlockSpec((_R, H), lambda i: (i, 0)),
            pl.BlockSpec((H, C), lambda i: (0, 0)),
            pl.BlockSpec((1, C), lambda i: (0, 0)),
        ],
        out_specs=pl.BlockSpec((_R, C), lambda i: (i, 0)),
        out_shape=jax.ShapeDtypeStruct((N, C), jnp.float32),
    )(s2, g2, dis, w2, b2)


def kernel(x, edge_index, W1, b1, W2, b2):
    ei = edge_index.astype(jnp.int32)
    src = ei[0].reshape(NS, KT, BB)
    dst = ei[1].reshape(NS, KT, BB)

    ones_rows = jnp.ones((BB, H), jnp.float32)
    zeros16 = jnp.zeros((RPT, H), jnp.float32)
    b1row = jnp.broadcast_to(b1.reshape(1, H), (16, H))

    h1 = _tc_matmul(x, W1)
    s1, g1, dis = _sc_a(h1, src, dst, ones_rows, zeros16)
    s2, g2 = _sc_b(s1, g1, dis, b1row, src, dst, zeros16)
    return _tc_final(s2, g2[0], dis[0], W2, b2.reshape(1, C))


# speedup vs baseline: 1.0683x; 1.0683x over previous
import functools

import jax
import jax.numpy as jnp
from jax import lax
from jax.experimental import pallas as pl
from jax.experimental.pallas import tpu as pltpu
from jax.experimental.pallas import tpu_sc as plsc

N = 10000
E = 320000
D = 128
H = 16
C = 40

NC = 2
NS = 16
KB = 125
BB = 80
KT = NC * KB
RPT = N // NS
P = 5
M = 2 * P

_mesh = plsc.VectorSubcoreMesh(core_axis_name="c", subcore_axis_name="s")
_sc_params = pltpu.CompilerParams(use_tc_tiling_on_sc=False,
                                  needs_layout_passes=False)


def _rsqrt16(x):
    i = plsc.bitcast(x, jnp.int32)
    i = 0x5F3759DF - lax.shift_right_logical(i, 1)
    y = plsc.bitcast(i, jnp.float32)
    xh = 0.5 * x
    for _ in range(3):
        y = y * (1.5 - xh * y * y)
    return y


def _sc_a(h1, src_blk, dst_blk, ones_rows, zero_rows):
    @functools.partial(
        pl.kernel,
        out_type=[
            jax.ShapeDtypeStruct((NC, N, H), jnp.float32),
            jax.ShapeDtypeStruct((NC, N, H), jnp.float32),
            jax.ShapeDtypeStruct((NC, N, H), jnp.float32),
        ],
        mesh=_mesh,
        compiler_params=_sc_params,
        scratch_types=[
            pltpu.VMEM((KT, BB), jnp.int32),
            pltpu.VMEM((KT, BB), jnp.int32),
            pltpu.VMEM((BB, H), jnp.float32),
            pltpu.VMEM((RPT, H), jnp.float32),
            pltpu.VMEM((RPT, H), jnp.float32),
            pltpu.VMEM((RPT, H), jnp.float32),
            pltpu.VMEM((M, BB, H), jnp.float32),
            pltpu.VMEM_SHARED((N, H), jnp.float32),
            pltpu.VMEM_SHARED((N, H), jnp.float32),
            pltpu.SemaphoreType.DMA((M,)),
            pltpu.SemaphoreType.DMA((M,)),
        ],
    )
    def sc_a(h1_hbm, src_hbm, dst_hbm, ones_hbm, zeros_hbm,
             s1_hbm, g1_hbm, dis_hbm,
             sidx, didx, ones_v, zv, va, vb, rows, acc, gtab, gsem, ssem):
        c = lax.axis_index("c")
        s = lax.axis_index("s")
        sl = pl.ds(s * RPT, RPT)
        pltpu.sync_copy(zeros_hbm, zv)
        pltpu.sync_copy(zv, acc.at[sl])
        pltpu.sync_copy(ones_hbm, ones_v)
        pltpu.sync_copy(src_hbm.at[s], sidx)
        pltpu.sync_copy(dst_hbm.at[s], didx)
        plsc.subcore_barrier()

        def deg_body(jj, _):
            for b in range(M):
                j = jj * M + b
                pltpu.async_copy(ones_v, acc.at[didx.at[j]], ssem.at[b],
                                 add=True)

                @pl.when(jj >= 1)
                def _wait():
                    pltpu.make_async_copy(
                        ones_v, acc.at[didx.at[j - M]], ssem.at[b]).wait()

            return _

        lax.fori_loop(0, KT // M, deg_body, None)
        for b in range(M):
            pltpu.make_async_copy(
                ones_v, acc.at[didx.at[KT - M + b]], ssem.at[b]).wait()
        plsc.subcore_barrier()

        pltpu.sync_copy(acc.at[sl], va)
        pltpu.sync_copy(h1_hbm.at[sl], vb)

        def dis_row(i, _):
            dis = _rsqrt16(va[i, :] + 1.0)
            va[i, :] = dis
            vb[i, :] = dis * vb[i, :]
            return _

        lax.fori_loop(0, RPT, dis_row, None)
        pltpu.sync_copy(va, dis_hbm.at[c, sl])
        pltpu.sync_copy(vb, g1_hbm.at[c, sl])
        pltpu.sync_copy(vb, gtab.at[sl])
        pltpu.sync_copy(zv, acc.at[sl])
        plsc.subcore_barrier()

        q0 = c * KB
        for b in range(P):
            pltpu.async_copy(gtab.at[sidx.at[q0 + b]], rows.at[b], gsem.at[b])

        def mp_body(jj, _):
            for b in range(M):
                jr = jj * M + b
                j = q0 + jr
                bp = (b + P) % M
                pltpu.make_async_copy(
                    gtab.at[sidx.at[j]], rows.at[b], gsem.at[b]).wait()
                pltpu.async_copy(rows.at[b], acc.at[didx.at[j]], ssem.at[b],
                                 add=True)

                @pl.when(jr + P < KB)
                def _next():
                    @pl.when(jr >= P)
                    def _retire():
                        pltpu.make_async_copy(
                            rows.at[bp], acc.at[didx.at[j - P]],
                            ssem.at[bp]).wait()

                    pltpu.async_copy(
                        gtab.at[sidx.at[j + P]], rows.at[bp], gsem.at[bp])

            return _

        lax.fori_loop(0, KB // M, mp_body, None)
        for b in range(P):
            jr = (KB // M) * M + b
            j = q0 + jr
            pltpu.make_async_copy(
                gtab.at[sidx.at[j]], rows.at[b], gsem.at[b]).wait()
            pltpu.async_copy(rows.at[b], acc.at[didx.at[j]], ssem.at[b],
                             add=True)
        for b in range(M):
            jr = KB - M + b
            pltpu.make_async_copy(
                rows.at[b], acc.at[didx.at[q0 + jr]], ssem.at[b]).wait()
        plsc.subcore_barrier()
        pltpu.sync_copy(acc.at[sl], va)
        pltpu.sync_copy(va, s1_hbm.at[c, sl])

    return sc_a(h1, src_blk, dst_blk, ones_rows, zero_rows)


def _sc_b(s1, g1, dis, b1row, src_blk, dst_blk, zero_rows):
    @functools.partial(
        pl.kernel,
        out_type=[
            jax.ShapeDtypeStruct((NC, N, H), jnp.float32),
            jax.ShapeDtypeStruct((NC, N, H), jnp.float32),
        ],
        mesh=_mesh,
        compiler_params=_sc_params,
        scratch_types=[
            pltpu.VMEM((KT, BB), jnp.int32),
            pltpu.VMEM((KT, BB), jnp.int32),
            pltpu.VMEM((16, H), jnp.float32),
            pltpu.VMEM((RPT, H), jnp.float32),
            pltpu.VMEM((RPT, H), jnp.float32),
            pltpu.VMEM((RPT, H), jnp.float32),
            pltpu.VMEM((RPT, H), jnp.float32),
            pltpu.VMEM((M, BB, H), jnp.float32),
            pltpu.VMEM_SHARED((N, H), jnp.float32),
            pltpu.VMEM_SHARED((N, H), jnp.float32),
            pltpu.SemaphoreType.DMA((M,)),
            pltpu.SemaphoreType.DMA((M,)),
        ],
    )
    def sc_b(s1_hbm, g1_hbm, dis_hbm, b1_hbm, src_hbm, dst_hbm, zeros_hbm,
             s2_hbm, g2_hbm,
             sidx, didx, b1v, zv, va, vb, vc, rows, acc, gtab, gsem, ssem):
        c = lax.axis_index("c")
        s = lax.axis_index("s")
        sl = pl.ds(s * RPT, RPT)
        pltpu.sync_copy(zeros_hbm, zv)
        pltpu.sync_copy(src_hbm.at[s], sidx)
        pltpu.sync_copy(dst_hbm.at[s], didx)
        pltpu.sync_copy(b1_hbm, b1v)

        pltpu.sync_copy(s1_hbm.at[0, sl], va)
        pltpu.sync_copy(s1_hbm.at[1, sl], vb)
        pltpu.sync_copy(g1_hbm.at[c, sl], vc)

        def s1_row(i, _):
            va[i, :] = va[i, :] + vb[i, :] + vc[i, :]
            return _

        lax.fori_loop(0, RPT, s1_row, None)
        pltpu.sync_copy(dis_hbm.at[c, sl], vb)

        def g2_row(i, _):
            agg = vb[i, :] * va[i, :] + b1v[0, :]
            va[i, :] = vb[i, :] * jnp.maximum(agg, 0.0)
            return _

        lax.fori_loop(0, RPT, g2_row, None)
        pltpu.sync_copy(va, g2_hbm.at[c, sl])
        pltpu.sync_copy(va, gtab.at[sl])
        pltpu.sync_copy(zv, acc.at[sl])
        plsc.subcore_barrier()

        q0 = c * KB
        for b in range(P):
            pltpu.async_copy(gtab.at[sidx.at[q0 + b]], rows.at[b], gsem.at[b])

        def mp_body(jj, _):
            for b in range(M):
                jr = jj * M + b
                j = q0 + jr
                bp = (b + P) % M
                pltpu.make_async_copy(
                    gtab.at[sidx.at[j]], rows.at[b], gsem.at[b]).wait()
                pltpu.async_copy(rows.at[b], acc.at[didx.at[j]], ssem.at[b],
                                 add=True)

                @pl.when(jr + P < KB)
                def _next():
                    @pl.when(jr >= P)
                    def _retire():
                        pltpu.make_async_copy(
                            rows.at[bp], acc.at[didx.at[j - P]],
                            ssem.at[bp]).wait()

                    pltpu.async_copy(
                        gtab.at[sidx.at[j + P]], rows.at[bp], gsem.at[bp])

            return _

        lax.fori_loop(0, KB // M, mp_body, None)
        for b in range(P):
            jr = (KB // M) * M + b
            j = q0 + jr
            pltpu.make_async_copy(
                gtab.at[sidx.at[j]], rows.at[b], gsem.at[b]).wait()
            pltpu.async_copy(rows.at[b], acc.at[didx.at[j]], ssem.at[b],
                             add=True)
        for b in range(M):
            jr = KB - M + b
            pltpu.make_async_copy(
                rows.at[b], acc.at[didx.at[q0 + jr]], ssem.at[b]).wait()
        plsc.subcore_barrier()
        pltpu.sync_copy(acc.at[sl], va)
        pltpu.sync_copy(va, s2_hbm.at[c, sl])

    return sc_b(s1, g1, dis, b1row, src_blk, dst_blk, zero_rows)


_R = 1000


def _tc_matmul(x, w1):
    def body(x_ref, w_ref, o_ref):
        o_ref[...] = jnp.dot(x_ref[...], w_ref[...],
                             preferred_element_type=jnp.float32)

    return pl.pallas_call(
        body,
        grid=(N // _R,),
        in_specs=[
            pl.BlockSpec((_R, D), lambda i: (i, 0)),
            pl.BlockSpec((D, H), lambda i: (0, 0)),
        ],
        out_specs=pl.BlockSpec((_R, H), lambda i: (i, 0)),
        out_shape=jax.ShapeDtypeStruct((N, H), jnp.float32),
    )(x, w1)


def _tc_final(s2, g2, dis, w2, b2):
    def body(s_ref, g_ref, dis_ref, w_ref, b_ref, o_ref):
        pre = jnp.dot(s_ref[0] + s_ref[1] + g_ref[...], w_ref[...],
                      preferred_element_type=jnp.float32)
        logits = dis_ref[:, :1] * pre + b_ref[...]
        m = jnp.max(logits, axis=1, keepdims=True)
        t = logits - m
        lse = jnp.log(jnp.sum(jnp.exp(t), axis=1, keepdims=True))
        o_ref[...] = t - lse

    return pl.pallas_call(
        body,
        grid=(N // _R,),
        in_specs=[
            pl.BlockSpec((NC, _R, H), lambda i: (0, i, 0)),
            pl.BlockSpec((_R, H), lambda i: (i, 0)),
            pl.BlockSpec((_R, H), lambda i: (i, 0)),
            pl.BlockSpec((H, C), lambda i: (0, 0)),
            pl.BlockSpec((1, C), lambda i: (0, 0)),
        ],
        out_specs=pl.BlockSpec((_R, C), lambda i: (i, 0)),
        out_shape=jax.ShapeDtypeStruct((N, C), jnp.float32),
    )(s2, g2, dis, w2, b2)


def kernel(x, edge_index, W1, b1, W2, b2):
    ei = edge_index.astype(jnp.int32)
    src = ei[0].reshape(NS, KT, BB)
    dst = ei[1].reshape(NS, KT, BB)

    ones_rows = jnp.ones((BB, H), jnp.float32)
    zeros16 = jnp.zeros((RPT, H), jnp.float32)
    b1row = jnp.broadcast_to(b1.reshape(1, H), (16, H))

    h1 = _tc_matmul(x, W1)
    s1, g1, dis = _sc_a(h1, src, dst, ones_rows, zeros16)
    s2, g2 = _sc_b(s1, g1, dis, b1row, src, dst, zeros16)
    return _tc_final(s2, g2[0], dis[0], W2, b2.reshape(1, C))


# speedup vs baseline: 1.2750x; 1.1935x over previous
import functools

import jax
import jax.numpy as jnp
from jax import lax
from jax.experimental import pallas as pl
from jax.experimental.pallas import tpu as pltpu
from jax.experimental.pallas import tpu_sc as plsc

N = 10000
E = 320000
D = 128
H = 16
C = 40

NC = 2
NS = 16
KB = 125
BB = 80
KT = NC * KB
RPT = N // NS
P = 5
M = 2 * P

_mesh = plsc.VectorSubcoreMesh(core_axis_name="c", subcore_axis_name="s")
_sc_params = pltpu.CompilerParams(use_tc_tiling_on_sc=False,
                                  needs_layout_passes=False)


def _rsqrt16(x):
    i = plsc.bitcast(x, jnp.int32)
    i = 0x5F3759DF - lax.shift_right_logical(i, 1)
    y = plsc.bitcast(i, jnp.float32)
    xh = 0.5 * x
    for _ in range(3):
        y = y * (1.5 - xh * y * y)
    return y


def _sc_a(h1, ei_blk, ones_rows, zero_rows):
    @functools.partial(
        pl.kernel,
        out_type=[
            jax.ShapeDtypeStruct((NC, N, H), jnp.float32),
            jax.ShapeDtypeStruct((NC, N, H), jnp.float32),
            jax.ShapeDtypeStruct((NC, N, H), jnp.float32),
        ],
        mesh=_mesh,
        compiler_params=_sc_params,
        scratch_types=[
            pltpu.VMEM((KT, BB), jnp.int32),
            pltpu.VMEM((KT, BB), jnp.int32),
            pltpu.VMEM((BB, H), jnp.float32),
            pltpu.VMEM((RPT, H), jnp.float32),
            pltpu.VMEM((RPT, H), jnp.float32),
            pltpu.VMEM((RPT, H), jnp.float32),
            pltpu.VMEM((M, BB, H), jnp.float32),
            pltpu.VMEM_SHARED((N, H), jnp.float32),
            pltpu.VMEM_SHARED((N, H), jnp.float32),
            pltpu.SemaphoreType.DMA((M,)),
            pltpu.SemaphoreType.DMA((M,)),
        ],
    )
    def sc_a(h1_hbm, ei_hbm, ones_hbm, zeros_hbm,
             s1_hbm, g1_hbm, dis_hbm,
             sidx, didx, ones_v, zv, va, vb, rows, acc, gtab, gsem, ssem):
        c = lax.axis_index("c")
        s = lax.axis_index("s")
        sl = pl.ds(s * RPT, RPT)
        pltpu.sync_copy(zeros_hbm, zv)
        pltpu.sync_copy(zv, acc.at[sl])
        pltpu.sync_copy(ones_hbm, ones_v)
        pltpu.sync_copy(ei_hbm.at[0, s], sidx)
        pltpu.sync_copy(ei_hbm.at[1, s], didx)
        plsc.subcore_barrier()

        def deg_body(jj, _):
            for b in range(M):
                j = jj * M + b
                pltpu.async_copy(ones_v, acc.at[didx.at[j]], ssem.at[b],
                                 add=True)

                @pl.when(jj >= 1)
                def _wait():
                    pltpu.make_async_copy(
                        ones_v, acc.at[didx.at[j - M]], ssem.at[b]).wait()

            return _

        lax.fori_loop(0, KT // M, deg_body, None)
        for b in range(M):
            pltpu.make_async_copy(
                ones_v, acc.at[didx.at[KT - M + b]], ssem.at[b]).wait()
        plsc.subcore_barrier()

        pltpu.sync_copy(acc.at[sl], va)
        pltpu.sync_copy(h1_hbm.at[sl], vb)

        def dis_row(i, _):
            dis = _rsqrt16(va[i, :] + 1.0)
            va[i, :] = dis
            vb[i, :] = dis * vb[i, :]
            return _

        lax.fori_loop(0, RPT, dis_row, None)
        pltpu.sync_copy(va, dis_hbm.at[c, sl])
        pltpu.sync_copy(vb, g1_hbm.at[c, sl])
        pltpu.sync_copy(vb, gtab.at[sl])
        pltpu.sync_copy(zv, acc.at[sl])
        plsc.subcore_barrier()

        q0 = c * KB
        for b in range(P):
            pltpu.async_copy(gtab.at[sidx.at[q0 + b]], rows.at[b], gsem.at[b])

        def mp_body(jj, _):
            for b in range(M):
                jr = jj * M + b
                j = q0 + jr
                bp = (b + P) % M
                pltpu.make_async_copy(
                    gtab.at[sidx.at[j]], rows.at[b], gsem.at[b]).wait()
                pltpu.async_copy(rows.at[b], acc.at[didx.at[j]], ssem.at[b],
                                 add=True)

                @pl.when(jr + P < KB)
                def _next():
                    @pl.when(jr >= P)
                    def _retire():
                        pltpu.make_async_copy(
                            rows.at[bp], acc.at[didx.at[j - P]],
                            ssem.at[bp]).wait()

                    pltpu.async_copy(
                        gtab.at[sidx.at[j + P]], rows.at[bp], gsem.at[bp])

            return _

        lax.fori_loop(0, KB // M, mp_body, None)
        for b in range(P):
            jr = (KB // M) * M + b
            j = q0 + jr
            pltpu.make_async_copy(
                gtab.at[sidx.at[j]], rows.at[b], gsem.at[b]).wait()
            pltpu.async_copy(rows.at[b], acc.at[didx.at[j]], ssem.at[b],
                             add=True)
        for b in range(M):
            jr = KB - M + b
            pltpu.make_async_copy(
                rows.at[b], acc.at[didx.at[q0 + jr]], ssem.at[b]).wait()
        plsc.subcore_barrier()
        pltpu.sync_copy(acc.at[sl], va)
        pltpu.sync_copy(va, s1_hbm.at[c, sl])

    return sc_a(h1, ei_blk, ones_rows, zero_rows)


def _sc_b(s1, g1, dis, b1row, ei_blk, zero_rows):
    @functools.partial(
        pl.kernel,
        out_type=jax.ShapeDtypeStruct((NC, N, H), jnp.float32),
        mesh=_mesh,
        compiler_params=_sc_params,
        scratch_types=[
            pltpu.VMEM((KT, BB), jnp.int32),
            pltpu.VMEM((KT, BB), jnp.int32),
            pltpu.VMEM((16, H), jnp.float32),
            pltpu.VMEM((RPT, H), jnp.float32),
            pltpu.VMEM((RPT, H), jnp.float32),
            pltpu.VMEM((RPT, H), jnp.float32),
            pltpu.VMEM((RPT, H), jnp.float32),
            pltpu.VMEM((M, BB, H), jnp.float32),
            pltpu.VMEM_SHARED((N, H), jnp.float32),
            pltpu.VMEM_SHARED((N, H), jnp.float32),
            pltpu.SemaphoreType.DMA((M,)),
            pltpu.SemaphoreType.DMA((M,)),
        ],
    )
    def sc_b(s1_hbm, g1_hbm, dis_hbm, b1_hbm, ei_hbm, zeros_hbm,
             s2_hbm,
             sidx, didx, b1v, zv, va, vb, vc, rows, acc, gtab, gsem, ssem):
        c = lax.axis_index("c")
        s = lax.axis_index("s")
        sl = pl.ds(s * RPT, RPT)
        pltpu.sync_copy(zeros_hbm, zv)
        pltpu.sync_copy(ei_hbm.at[0, s], sidx)
        pltpu.sync_copy(ei_hbm.at[1, s], didx)
        pltpu.sync_copy(b1_hbm, b1v)

        pltpu.sync_copy(s1_hbm.at[0, sl], va)
        pltpu.sync_copy(s1_hbm.at[1, sl], vb)
        pltpu.sync_copy(g1_hbm.at[c, sl], vc)

        def s1_row(i, _):
            va[i, :] = va[i, :] + vb[i, :] + vc[i, :]
            return _

        lax.fori_loop(0, RPT, s1_row, None)
        pltpu.sync_copy(dis_hbm.at[c, sl], vb)

        def g2_row(i, _):
            agg = vb[i, :] * va[i, :] + b1v[0, :]
            va[i, :] = vb[i, :] * jnp.maximum(agg, 0.0)
            return _

        lax.fori_loop(0, RPT, g2_row, None)
        pltpu.sync_copy(va, gtab.at[sl])

        @pl.when(c == 0)
        def _seed():
            pltpu.sync_copy(va, acc.at[sl])

        @pl.when(c != 0)
        def _zero():
            pltpu.sync_copy(zv, acc.at[sl])

        plsc.subcore_barrier()

        q0 = c * KB
        for b in range(P):
            pltpu.async_copy(gtab.at[sidx.at[q0 + b]], rows.at[b], gsem.at[b])

        def mp_body(jj, _):
            for b in range(M):
                jr = jj * M + b
                j = q0 + jr
                bp = (b + P) % M
                pltpu.make_async_copy(
                    gtab.at[sidx.at[j]], rows.at[b], gsem.at[b]).wait()
                pltpu.async_copy(rows.at[b], acc.at[didx.at[j]], ssem.at[b],
                                 add=True)

                @pl.when(jr + P < KB)
                def _next():
                    @pl.when(jr >= P)
                    def _retire():
                        pltpu.make_async_copy(
                            rows.at[bp], acc.at[didx.at[j - P]],
                            ssem.at[bp]).wait()

                    pltpu.async_copy(
                        gtab.at[sidx.at[j + P]], rows.at[bp], gsem.at[bp])

            return _

        lax.fori_loop(0, KB // M, mp_body, None)
        for b in range(P):
            jr = (KB // M) * M + b
            j = q0 + jr
            pltpu.make_async_copy(
                gtab.at[sidx.at[j]], rows.at[b], gsem.at[b]).wait()
            pltpu.async_copy(rows.at[b], acc.at[didx.at[j]], ssem.at[b],
                             add=True)
        for b in range(M):
            jr = KB - M + b
            pltpu.make_async_copy(
                rows.at[b], acc.at[didx.at[q0 + jr]], ssem.at[b]).wait()
        plsc.subcore_barrier()
        pltpu.sync_copy(acc.at[sl], va)

        def scale_row(i, _):
            va[i, :] = vb[i, :] * va[i, :]
            return _

        lax.fori_loop(0, RPT, scale_row, None)
        pltpu.sync_copy(va, s2_hbm.at[c, sl])

    return sc_b(s1, g1, dis, b1row, ei_blk, zero_rows)


_R = 1000
_RM = 2000


def _tc_matmul(x, w1):
    def body(x_ref, w_ref, o_ref):
        o_ref[...] = jnp.dot(x_ref[...], w_ref[...],
                             preferred_element_type=jnp.float32)

    return pl.pallas_call(
        body,
        grid=(N // _RM,),
        in_specs=[
            pl.BlockSpec((_RM, D), lambda i: (i, 0)),
            pl.BlockSpec((D, H), lambda i: (0, 0)),
        ],
        out_specs=pl.BlockSpec((_RM, H), lambda i: (i, 0)),
        out_shape=jax.ShapeDtypeStruct((N, H), jnp.float32),
    )(x, w1)


def _tc_final(s2, w2, b2):
    def body(s_ref, w_ref, b_ref, o_ref):
        logits = jnp.dot(s_ref[0] + s_ref[1], w_ref[...],
                         preferred_element_type=jnp.float32) + b_ref[...]
        m = jnp.max(logits, axis=1, keepdims=True)
        t = logits - m
        lse = jnp.log(jnp.sum(jnp.exp(t), axis=1, keepdims=True))
        o_ref[...] = t - lse

    return pl.pallas_call(
        body,
        grid=(N // _R,),
        in_specs=[
            pl.BlockSpec((NC, _R, H), lambda i: (0, i, 0)),
            pl.BlockSpec((H, C), lambda i: (0, 0)),
            pl.BlockSpec((1, C), lambda i: (0, 0)),
        ],
        out_specs=pl.BlockSpec((_R, C), lambda i: (i, 0)),
        out_shape=jax.ShapeDtypeStruct((N, C), jnp.float32),
    )(s2, w2, b2)


def kernel(x, edge_index, W1, b1, W2, b2):
    ei = edge_index.astype(jnp.int32).reshape(2, NS, KT, BB)

    ones_rows = jnp.ones((BB, H), jnp.float32)
    zeros16 = jnp.zeros((RPT, H), jnp.float32)
    b1row = jnp.broadcast_to(b1.reshape(1, H), (16, H))

    h1 = _tc_matmul(x, W1)
    s1, g1, dis = _sc_a(h1, ei, ones_rows, zeros16)
    s2 = _sc_b(s1, g1, dis, b1row, ei, zeros16)
    return _tc_final(s2, W2, b2.reshape(1, C))


# speedup vs baseline: 1.3322x; 1.0449x over previous
import functools

import jax
import jax.numpy as jnp
from jax import lax
from jax.experimental import pallas as pl
from jax.experimental.pallas import tpu as pltpu
from jax.experimental.pallas import tpu_sc as plsc

N = 10000
E = 320000
D = 128
H = 16
C = 40

NC = 2
NS = 16
KB = 125
BB = 80
KT = NC * KB
RPT = N // NS
P = 5
M = 2 * P

_mesh = plsc.VectorSubcoreMesh(core_axis_name="c", subcore_axis_name="s")
_sc_params = pltpu.CompilerParams(use_tc_tiling_on_sc=False,
                                  needs_layout_passes=False)


def _rsqrt16(x):
    i = plsc.bitcast(x, jnp.int32)
    i = 0x5F3759DF - lax.shift_right_logical(i, 1)
    y = plsc.bitcast(i, jnp.float32)
    xh = 0.5 * x
    for _ in range(3):
        y = y * (1.5 - xh * y * y)
    return y


def _sc_a(h1, ei_blk, ones_rows, zero_rows):
    @functools.partial(
        pl.kernel,
        out_type=[
            jax.ShapeDtypeStruct((NC, N, H), jnp.float32),
            jax.ShapeDtypeStruct((NC, N, H), jnp.float32),
        ],
        mesh=_mesh,
        compiler_params=_sc_params,
        scratch_types=[
            pltpu.VMEM((KT, BB), jnp.int32),
            pltpu.VMEM((KT, BB), jnp.int32),
            pltpu.VMEM((BB, H), jnp.float32),
            pltpu.VMEM((RPT, H), jnp.float32),
            pltpu.VMEM((RPT, H), jnp.float32),
            pltpu.VMEM((RPT, H), jnp.float32),
            pltpu.VMEM((M, BB, H), jnp.float32),
            pltpu.VMEM_SHARED((N, H), jnp.float32),
            pltpu.VMEM_SHARED((N, H), jnp.float32),
            pltpu.SemaphoreType.DMA((M,)),
            pltpu.SemaphoreType.DMA((M,)),
        ],
    )
    def sc_a(h1_hbm, ei_hbm, ones_hbm, zeros_hbm,
             s1_hbm, dis_hbm,
             sidx, didx, ones_v, zv, va, vb, rows, acc, gtab, gsem, ssem):
        c = lax.axis_index("c")
        s = lax.axis_index("s")
        sl = pl.ds(s * RPT, RPT)
        pltpu.sync_copy(zeros_hbm, zv)
        pltpu.sync_copy(zv, acc.at[sl])
        pltpu.sync_copy(ones_hbm, ones_v)
        pltpu.sync_copy(ei_hbm.at[0, s], sidx)
        pltpu.sync_copy(ei_hbm.at[1, s], didx)
        plsc.subcore_barrier()

        def deg_body(jj, _):
            for b in range(M):
                j = jj * M + b
                pltpu.async_copy(ones_v, acc.at[didx.at[j]], ssem.at[b],
                                 add=True)

                @pl.when(jj >= 1)
                def _wait():
                    pltpu.make_async_copy(
                        ones_v, acc.at[didx.at[j - M]], ssem.at[b]).wait()

            return _

        lax.fori_loop(0, KT // M, deg_body, None)
        for b in range(M):
            pltpu.make_async_copy(
                ones_v, acc.at[didx.at[KT - M + b]], ssem.at[b]).wait()
        plsc.subcore_barrier()

        pltpu.sync_copy(acc.at[sl], va)
        pltpu.sync_copy(h1_hbm.at[sl], vb)

        def dis_row(i, _):
            dis = _rsqrt16(va[i, :] + 1.0)
            va[i, :] = dis
            vb[i, :] = dis * vb[i, :]
            return _

        lax.fori_loop(0, RPT, dis_row, None)
        pltpu.sync_copy(va, dis_hbm.at[c, sl])
        pltpu.sync_copy(vb, gtab.at[sl])

        @pl.when(c == 0)
        def _seed():
            pltpu.sync_copy(vb, acc.at[sl])

        @pl.when(c != 0)
        def _zero():
            pltpu.sync_copy(zv, acc.at[sl])

        plsc.subcore_barrier()

        q0 = c * KB
        for b in range(P):
            pltpu.async_copy(gtab.at[sidx.at[q0 + b]], rows.at[b], gsem.at[b])

        def mp_body(jj, _):
            for b in range(M):
                jr = jj * M + b
                j = q0 + jr
                bp = (b + P) % M
                pltpu.make_async_copy(
                    gtab.at[sidx.at[j]], rows.at[b], gsem.at[b]).wait()
                pltpu.async_copy(rows.at[b], acc.at[didx.at[j]], ssem.at[b],
                                 add=True)

                @pl.when(jr + P < KB)
                def _next():
                    @pl.when(jr >= P)
                    def _retire():
                        pltpu.make_async_copy(
                            rows.at[bp], acc.at[didx.at[j - P]],
                            ssem.at[bp]).wait()

                    pltpu.async_copy(
                        gtab.at[sidx.at[j + P]], rows.at[bp], gsem.at[bp])

            return _

        lax.fori_loop(0, KB // M, mp_body, None)
        for b in range(P):
            jr = (KB // M) * M + b
            j = q0 + jr
            pltpu.make_async_copy(
                gtab.at[sidx.at[j]], rows.at[b], gsem.at[b]).wait()
            pltpu.async_copy(rows.at[b], acc.at[didx.at[j]], ssem.at[b],
                             add=True)
        for b in range(M):
            jr = KB - M + b
            pltpu.make_async_copy(
                rows.at[b], acc.at[didx.at[q0 + jr]], ssem.at[b]).wait()
        plsc.subcore_barrier()
        pltpu.sync_copy(acc.at[sl], va)
        pltpu.sync_copy(va, s1_hbm.at[c, sl])

    return sc_a(h1, ei_blk, ones_rows, zero_rows)


def _sc_b(s1, dis, b1row, ei_blk, zero_rows):
    @functools.partial(
        pl.kernel,
        out_type=jax.ShapeDtypeStruct((NC, N, H), jnp.float32),
        mesh=_mesh,
        compiler_params=_sc_params,
        scratch_types=[
            pltpu.VMEM((KT, BB), jnp.int32),
            pltpu.VMEM((KT, BB), jnp.int32),
            pltpu.VMEM((16, H), jnp.float32),
            pltpu.VMEM((RPT, H), jnp.float32),
            pltpu.VMEM((RPT, H), jnp.float32),
            pltpu.VMEM((RPT, H), jnp.float32),
            pltpu.VMEM((RPT, H), jnp.float32),
            pltpu.VMEM((M, BB, H), jnp.float32),
            pltpu.VMEM_SHARED((N, H), jnp.float32),
            pltpu.VMEM_SHARED((N, H), jnp.float32),
            pltpu.SemaphoreType.DMA((M,)),
            pltpu.SemaphoreType.DMA((M,)),
        ],
    )
    def sc_b(s1_hbm, dis_hbm, b1_hbm, ei_hbm, zeros_hbm,
             s2_hbm,
             sidx, didx, b1v, zv, va, vb, vc, rows, acc, gtab, gsem, ssem):
        c = lax.axis_index("c")
        s = lax.axis_index("s")
        sl = pl.ds(s * RPT, RPT)
        pltpu.sync_copy(zeros_hbm, zv)
        pltpu.sync_copy(ei_hbm.at[0, s], sidx)
        pltpu.sync_copy(ei_hbm.at[1, s], didx)
        pltpu.sync_copy(b1_hbm, b1v)

        pltpu.sync_copy(s1_hbm.at[0, sl], va)
        pltpu.sync_copy(s1_hbm.at[1, sl], vc)
        pltpu.sync_copy(dis_hbm.at[c, sl], vb)

        def g2_row(i, _):
            agg = vb[i, :] * (va[i, :] + vc[i, :]) + b1v[0, :]
            va[i, :] = vb[i, :] * jnp.maximum(agg, 0.0)
            return _

        lax.fori_loop(0, RPT, g2_row, None)
        pltpu.sync_copy(va, gtab.at[sl])

        @pl.when(c == 0)
        def _seed():
            pltpu.sync_copy(va, acc.at[sl])

        @pl.when(c != 0)
        def _zero():
            pltpu.sync_copy(zv, acc.at[sl])

        plsc.subcore_barrier()

        q0 = c * KB
        for b in range(P):
            pltpu.async_copy(gtab.at[sidx.at[q0 + b]], rows.at[b], gsem.at[b])

        def mp_body(jj, _):
            for b in range(M):
                jr = jj * M + b
                j = q0 + jr
                bp = (b + P) % M
                pltpu.make_async_copy(
                    gtab.at[sidx.at[j]], rows.at[b], gsem.at[b]).wait()
                pltpu.async_copy(rows.at[b], acc.at[didx.at[j]], ssem.at[b],
                                 add=True)

                @pl.when(jr + P < KB)
                def _next():
                    @pl.when(jr >= P)
                    def _retire():
                        pltpu.make_async_copy(
                            rows.at[bp], acc.at[didx.at[j - P]],
                            ssem.at[bp]).wait()

                    pltpu.async_copy(
                        gtab.at[sidx.at[j + P]], rows.at[bp], gsem.at[bp])

            return _

        lax.fori_loop(0, KB // M, mp_body, None)
        for b in range(P):
            jr = (KB // M) * M + b
            j = q0 + jr
            pltpu.make_async_copy(
                gtab.at[sidx.at[j]], rows.at[b], gsem.at[b]).wait()
            pltpu.async_copy(rows.at[b], acc.at[didx.at[j]], ssem.at[b],
                             add=True)
        for b in range(M):
            jr = KB - M + b
            pltpu.make_async_copy(
                rows.at[b], acc.at[didx.at[q0 + jr]], ssem.at[b]).wait()
        plsc.subcore_barrier()
        pltpu.sync_copy(acc.at[sl], va)

        def scale_row(i, _):
            va[i, :] = vb[i, :] * va[i, :]
            return _

        lax.fori_loop(0, RPT, scale_row, None)
        pltpu.sync_copy(va, s2_hbm.at[c, sl])

    return sc_b(s1, dis, b1row, ei_blk, zero_rows)


_R = 1000
_RM = 2000


def _tc_matmul(x, w1):
    def body(x_ref, w_ref, o_ref):
        o_ref[...] = jnp.dot(x_ref[...], w_ref[...],
                             preferred_element_type=jnp.float32)

    return pl.pallas_call(
        body,
        grid=(N // _RM,),
        in_specs=[
            pl.BlockSpec((_RM, D), lambda i: (i, 0)),
            pl.BlockSpec((D, H), lambda i: (0, 0)),
        ],
        out_specs=pl.BlockSpec((_RM, H), lambda i: (i, 0)),
        out_shape=jax.ShapeDtypeStruct((N, H), jnp.float32),
    )(x, w1)


def _tc_final(s2, w2, b2):
    def body(s_ref, w_ref, b_ref, o_ref):
        logits = jnp.dot(s_ref[0] + s_ref[1], w_ref[...],
                         preferred_element_type=jnp.float32) + b_ref[...]
        m = jnp.max(logits, axis=1, keepdims=True)
        t = logits - m
        lse = jnp.log(jnp.sum(jnp.exp(t), axis=1, keepdims=True))
        o_ref[...] = t - lse

    return pl.pallas_call(
        body,
        grid=(N // _RM,),
        in_specs=[
            pl.BlockSpec((NC, _RM, H), lambda i: (0, i, 0)),
            pl.BlockSpec((H, C), lambda i: (0, 0)),
            pl.BlockSpec((1, C), lambda i: (0, 0)),
        ],
        out_specs=pl.BlockSpec((_RM, C), lambda i: (i, 0)),
        out_shape=jax.ShapeDtypeStruct((N, C), jnp.float32),
    )(s2, w2, b2)


def kernel(x, edge_index, W1, b1, W2, b2):
    ei = edge_index.astype(jnp.int32).reshape(2, NS, KT, BB)

    ones_rows = jnp.ones((BB, H), jnp.float32)
    zeros16 = jnp.zeros((RPT, H), jnp.float32)
    b1row = jnp.broadcast_to(b1.reshape(1, H), (16, H))

    h1 = _tc_matmul(x, W1)
    s1, dis = _sc_a(h1, ei, ones_rows, zeros16)
    s2 = _sc_b(s1, dis, b1row, ei, zeros16)
    return _tc_final(s2, W2, b2.reshape(1, C))
